# baseline (speedup 1.0000x reference)
import math

import jax
import jax.numpy as jnp
from jax.experimental import pallas as pl
from jax.experimental.pallas import tpu as pltpu

N_CHANNELS = 128
RATIO = 0.5


def _score_body(w_ref, p_ref, nrm_ref, out_ref):
    logits = jax.lax.dot_general(
        w_ref[...], p_ref[...],
        dimension_numbers=(((1,), (0,)), ((), ())),
        preferred_element_type=jnp.float32,
    )
    out_ref[...] = jnp.tanh(logits / nrm_ref[0, 0])


def _scores(weights, p):
    total = weights.shape[0]
    blk = 1024
    nrm = jnp.linalg.norm(p).reshape(1, 1)
    wb = weights.astype(jnp.bfloat16)
    pb = p.astype(jnp.bfloat16).reshape(N_CHANNELS, 1)
    grid = (total // blk,)
    return pl.pallas_call(
        _score_body,
        grid=grid,
        in_specs=[
            pl.BlockSpec((blk, N_CHANNELS), lambda i: (i, 0)),
            pl.BlockSpec((N_CHANNELS, 1), lambda i: (0, 0)),
            pl.BlockSpec(memory_space=pltpu.SMEM),
        ],
        out_specs=pl.BlockSpec((blk, 1), lambda i: (i, 0)),
        out_shape=jax.ShapeDtypeStruct((total, 1), jnp.float32),
    )(wb, pb, nrm).reshape(total)


def kernel(positions, weights, batch, p):
    nb = batch.shape[0]
    total = positions.shape[0]
    n_per = total // nb
    k = int(math.ceil(RATIO * n_per))

    score = _scores(weights, p)

    dense = score.reshape(nb, n_per)
    perm = jnp.argsort(-dense, axis=1)[:, :k]
    offsets = (jnp.arange(nb, dtype=jnp.int32) * n_per)[:, None]
    node_index = (offsets + perm).reshape(-1)

    w = score[node_index]
    pos_sel = jnp.take(positions, node_index, axis=0)
    w_sel = jnp.take(weights, node_index, axis=0) * w[:, None]
    new_batch = jnp.full((nb,), k, dtype=jnp.int32)
    return pos_sel, w_sel, new_batch
